# baseline (speedup 1.0000x reference)
import functools

import jax
import jax.numpy as jnp
from jax import lax
from jax.experimental import pallas as pl
from jax.experimental.pallas import tpu as pltpu
from jax.experimental.pallas import tpu_sc as plsc

NUM_EMBEDDINGS = 100000
EMBEDDING_DIM = 128
BATCH = 16384
HIST = 50

NC = 2
NS = 16
NW = NC * NS

B_TOTAL = BATCH * HIST
B_PER_W = B_TOTAL // NW
CHUNK = 128
NCHUNK = B_PER_W // CHUNK
NBUF = 5
LEAD = 2

_mesh = plsc.VectorSubcoreMesh(
    core_axis_name="c", subcore_axis_name="s", num_cores=NC, num_subcores=NS
)


@functools.partial(
    pl.kernel,
    out_type=jax.ShapeDtypeStruct((B_TOTAL, EMBEDDING_DIM), jnp.float32),
    mesh=_mesh,
    scratch_types=[
        pltpu.VMEM((NCHUNK, CHUNK), jnp.int32),
        [pltpu.VMEM((CHUNK, EMBEDDING_DIM), jnp.float32) for _ in range(NBUF)],
        [pltpu.SemaphoreType.DMA for _ in range(NBUF)],
        [pltpu.SemaphoreType.DMA for _ in range(NBUF)],
    ],
)
def _sc_gather(idx_hbm, table_hbm, out_hbm, idx_v, rows, gsem, wsem):
    wid = lax.axis_index("s") * NC + lax.axis_index("c")
    base = wid * B_PER_W
    pltpu.sync_copy(idx_hbm.at[wid], idx_v)

    for b in range(LEAD):
        pltpu.async_copy(table_hbm.at[idx_v.at[b]], rows[b], gsem[b])

    @pl.loop(0, NCHUNK, step=NBUF)
    def _group(j):
        for b in range(NBUF):
            c = j + b
            nxt = c + LEAD
            nb = (b + LEAD) % NBUF

            @pl.when(nxt < NCHUNK)
            def _issue_gather():
                @pl.when(nxt >= NBUF)
                def _reclaim():
                    prev = nxt - NBUF
                    pltpu.make_async_copy(
                        rows[nb],
                        out_hbm.at[pl.ds(base + prev * CHUNK, CHUNK)],
                        wsem[nb],
                    ).wait()

                pltpu.async_copy(table_hbm.at[idx_v.at[nxt]], rows[nb], gsem[nb])

            pltpu.make_async_copy(
                table_hbm.at[idx_v.at[c]], rows[b], gsem[b]
            ).wait()
            pltpu.async_copy(
                rows[b], out_hbm.at[pl.ds(base + c * CHUNK, CHUNK)], wsem[b]
            )

    for b in range(NBUF):
        j_last = NCHUNK - NBUF + b
        pltpu.make_async_copy(
            rows[b], out_hbm.at[pl.ds(base + j_last * CHUNK, CHUNK)], wsem[b]
        ).wait()


def kernel(labels, weight):
    idx = labels.reshape(NW, NCHUNK, CHUNK)
    out = _sc_gather(idx, weight)
    return out.reshape(BATCH, HIST, EMBEDDING_DIM)


# speedup vs baseline: 1.8435x; 1.8435x over previous
import functools

import jax
import jax.numpy as jnp
from jax import lax
from jax.experimental import pallas as pl
from jax.experimental.pallas import tpu as pltpu
from jax.experimental.pallas import tpu_sc as plsc

NUM_EMBEDDINGS = 100000
EMBEDDING_DIM = 128
BATCH = 16384
HIST = 50

NC = 2
NS = 16
NW = NC * NS

ROWS_PER_W = BATCH // NW
NBUF = 8
LEAD = 3

_mesh = plsc.VectorSubcoreMesh(
    core_axis_name="c", subcore_axis_name="s", num_cores=NC, num_subcores=NS
)


@functools.partial(
    pl.kernel,
    out_type=jax.ShapeDtypeStruct((BATCH, HIST, EMBEDDING_DIM), jnp.float32),
    mesh=_mesh,
    scratch_types=[
        pltpu.VMEM((ROWS_PER_W, HIST), jnp.int32),
        [pltpu.VMEM((HIST, EMBEDDING_DIM), jnp.float32) for _ in range(NBUF)],
        [pltpu.SemaphoreType.DMA for _ in range(NBUF)],
        [pltpu.SemaphoreType.DMA for _ in range(NBUF)],
    ],
)
def _sc_gather(lab_hbm, table_hbm, out_hbm, idx_v, rows, gsem, wsem):
    wid = lax.axis_index("s") * NC + lax.axis_index("c")
    base = wid * ROWS_PER_W
    pltpu.sync_copy(lab_hbm.at[pl.ds(base, ROWS_PER_W)], idx_v)

    for b in range(LEAD):
        pltpu.async_copy(table_hbm.at[idx_v.at[b]], rows[b], gsem[b])

    @pl.loop(0, ROWS_PER_W, step=NBUF)
    def _group(j):
        for b in range(NBUF):
            c = j + b
            nxt = c + LEAD
            nb = (b + LEAD) % NBUF

            @pl.when(nxt < ROWS_PER_W)
            def _issue_gather():
                @pl.when(nxt >= NBUF)
                def _reclaim():
                    prev = nxt - NBUF
                    pltpu.make_async_copy(
                        rows[nb], out_hbm.at[base + prev], wsem[nb]
                    ).wait()

                pltpu.async_copy(table_hbm.at[idx_v.at[nxt]], rows[nb], gsem[nb])

            pltpu.make_async_copy(
                table_hbm.at[idx_v.at[c]], rows[b], gsem[b]
            ).wait()
            pltpu.async_copy(rows[b], out_hbm.at[base + c], wsem[b])

    for b in range(NBUF):
        j_last = ROWS_PER_W - NBUF + b
        pltpu.make_async_copy(
            rows[b], out_hbm.at[base + j_last], wsem[b]
        ).wait()


def kernel(labels, weight):
    return _sc_gather(labels, weight)


# speedup vs baseline: 3.5976x; 1.9516x over previous
import functools

import jax
import jax.numpy as jnp
from jax import lax
from jax.experimental import pallas as pl
from jax.experimental.pallas import tpu as pltpu
from jax.experimental.pallas import tpu_sc as plsc

NUM_EMBEDDINGS = 100000
EMBEDDING_DIM = 128
BATCH = 16384
HIST = 50

NC = 2
NS = 16
NW = NC * NS

ROWS_PER_W = BATCH // NW
CHUNK = 128
KPH = ROWS_PER_W // CHUNK
NCHUNK = HIST * KPH
NBUF = 5
LEAD = 2

_mesh = plsc.VectorSubcoreMesh(
    core_axis_name="c", subcore_axis_name="s", num_cores=NC, num_subcores=NS
)


@functools.partial(
    pl.kernel,
    out_type=jax.ShapeDtypeStruct((HIST, BATCH, EMBEDDING_DIM), jnp.float32),
    mesh=_mesh,
    scratch_types=[
        pltpu.VMEM((HIST, ROWS_PER_W), jnp.int32),
        [pltpu.VMEM((CHUNK, EMBEDDING_DIM), jnp.float32) for _ in range(NBUF)],
        [pltpu.SemaphoreType.DMA for _ in range(NBUF)],
        [pltpu.SemaphoreType.DMA for _ in range(NBUF)],
    ],
)
def _sc_gather(lab_hbm, table_hbm, out_hbm, idx_v, rows, gsem, wsem):
    wid = lax.axis_index("s") * NC + lax.axis_index("c")
    base = wid * ROWS_PER_W
    pltpu.sync_copy(lab_hbm.at[:, pl.ds(base, ROWS_PER_W)], idx_v)

    def idx_slice(c):
        return idx_v.at[c // KPH, pl.ds((c % KPH) * CHUNK, CHUNK)]

    def out_slice(c):
        return out_hbm.at[c // KPH, pl.ds(base + (c % KPH) * CHUNK, CHUNK)]

    for b in range(LEAD):
        pltpu.async_copy(table_hbm.at[idx_slice(b)], rows[b], gsem[b])

    @pl.loop(0, NCHUNK, step=NBUF)
    def _group(j):
        for b in range(NBUF):
            c = j + b
            nxt = c + LEAD
            nb = (b + LEAD) % NBUF

            @pl.when(nxt < NCHUNK)
            def _issue_gather():
                @pl.when(nxt >= NBUF)
                def _reclaim():
                    pltpu.make_async_copy(
                        rows[nb], out_slice(nxt - NBUF), wsem[nb]
                    ).wait()

                pltpu.async_copy(table_hbm.at[idx_slice(nxt)], rows[nb], gsem[nb])

            pltpu.make_async_copy(
                table_hbm.at[idx_slice(c)], rows[b], gsem[b]
            ).wait()
            pltpu.async_copy(rows[b], out_slice(c), wsem[b])

    for b in range(NBUF):
        pltpu.make_async_copy(
            rows[b], out_slice(NCHUNK - NBUF + b), wsem[b]
        ).wait()


def kernel(labels, weight):
    out_t = _sc_gather(labels.T, weight)
    return jnp.transpose(out_t, (1, 0, 2))
